# baseline (speedup 1.0000x reference)
import jax
import jax.numpy as jnp
from jax.experimental import pallas as pl
from jax.experimental.pallas import tpu as pltpu

N_G, N_N, N_F = 128, 32, 64
E_FEAT, EMB, PASSES = 16, 64, 4
P_TILE = 4
HALF = N_G // 2


def _body(na_ref, nb_ref, ea_ref, eb_ref, wpre_ref, wmsg_ref, wout_ref,
          oa_ref, ob_ref):
    f32, bf = jnp.float32, jnp.bfloat16

    def _dot(a, b):
        return jnp.dot(a.astype(bf), b.astype(bf),
                       preferred_element_type=f32)

    nodes = jnp.concatenate([na_ref[...], nb_ref[...]], -1)
    edges = jnp.concatenate([ea_ref[...], eb_ref[...]], -1)

    def _bd(w, rows):
        z = jnp.zeros((rows, EMB), f32)
        return jnp.concatenate(
            [jnp.concatenate([w, z], 1), jnp.concatenate([z, w], 1)], 0)

    w_src2 = _bd(wpre_ref[:N_F, :], N_F)
    w_dst2 = _bd(wpre_ref[N_F:2 * N_F, :], N_F)
    w_e2 = _bd(wpre_ref[2 * N_F:, :], E_FEAT)
    w_msg2 = _bd(wmsg_ref[...], EMB)
    w_out2 = _bd(wout_ref[...], EMB)
    sel = _bd(jnp.ones((E_FEAT, EMB), f32), E_FEAT)

    n_e = P_TILE * N_N * N_N
    shp = (P_TILE, N_N, N_N, 2 * EMB)
    adj_e = _dot(edges.reshape(n_e, 2 * E_FEAT), sel).reshape(shp)
    adj_t_e = jnp.swapaxes(adj_e, 1, 2)

    a_src = _dot(nodes.reshape(P_TILE * N_N, 2 * N_F), w_src2).reshape(
        P_TILE, N_N, 2 * EMB)
    a_dst = _dot(nodes.reshape(P_TILE * N_N, 2 * N_F), w_dst2).reshape(
        P_TILE, N_N, 2 * EMB)
    e_term = _dot(edges.reshape(n_e, 2 * E_FEAT), w_e2).reshape(shp)
    ef = jnp.tanh(a_src[:, :, None, :] + a_dst[:, None, :, :] + e_term)
    ef_t = jnp.swapaxes(ef, 1, 2)

    def _mm(x):
        return _dot(x.reshape(n_e, 2 * EMB), w_msg2).reshape(shp)

    def _pass(x_bf, mask_e, ef_out, reduce_axis):
        xr = x_bf.astype(f32)
        masked = xr * mask_e
        s = masked.sum(axis=reduce_axis, keepdims=True)
        return jnp.tanh(ef_out + _mm(s - masked))

    y = jnp.tanh(ef_t).astype(bf)
    x = _pass(y, adj_t_e, ef, reduce_axis=1).astype(bf)
    y = _pass(x, adj_e, ef_t, reduce_axis=2).astype(bf)
    x = _pass(y, adj_t_e, ef, reduce_axis=1)

    graph = (x * adj_e).sum(axis=1).sum(axis=1)
    out = jnp.tanh(_dot(graph, w_out2))
    oa_ref[0, :, :] = out[:, :EMB]
    ob_ref[0, :, :] = out[:, EMB:]


def kernel(nodes, edges, W_pre, W_msg, W_out):
    grid = (HALF // P_TILE,)
    shift = HALF // P_TILE
    out_a, out_b = pl.pallas_call(
        _body,
        grid=grid,
        in_specs=[
            pl.BlockSpec((P_TILE, N_N, N_F), lambda i: (i, 0, 0)),
            pl.BlockSpec((P_TILE, N_N, N_F), lambda i: (i + shift, 0, 0)),
            pl.BlockSpec((P_TILE, N_N, N_N, E_FEAT), lambda i: (i, 0, 0, 0)),
            pl.BlockSpec((P_TILE, N_N, N_N, E_FEAT),
                         lambda i: (i + shift, 0, 0, 0)),
            pl.BlockSpec((2 * N_F + E_FEAT, EMB), lambda i: (0, 0)),
            pl.BlockSpec((EMB, EMB), lambda i: (0, 0)),
            pl.BlockSpec((EMB, EMB), lambda i: (0, 0)),
        ],
        out_specs=[
            pl.BlockSpec((1, P_TILE, EMB), lambda i: (i, 0, 0)),
            pl.BlockSpec((1, P_TILE, EMB), lambda i: (i, 0, 0)),
        ],
        out_shape=[
            jax.ShapeDtypeStruct((shift, P_TILE, EMB), jnp.float32),
            jax.ShapeDtypeStruct((shift, P_TILE, EMB), jnp.float32),
        ],
        compiler_params=pltpu.CompilerParams(
            dimension_semantics=("parallel",)),
    )(nodes, nodes, edges, edges, W_pre, W_msg, W_out)
    return jnp.concatenate([out_a, out_b], axis=0).reshape(N_G, EMB)


# speedup vs baseline: 1.0087x; 1.0087x over previous
import jax
import jax.numpy as jnp
from jax.experimental import pallas as pl
from jax.experimental.pallas import tpu as pltpu

N_G, N_N, N_F = 128, 32, 64
E_FEAT, EMB, PASSES = 16, 64, 4
P_TILE = 8
HALF = N_G // 2


def _body(na_ref, nb_ref, ea_ref, eb_ref, wpre_ref, wmsg_ref, wout_ref,
          oa_ref, ob_ref):
    f32, bf = jnp.float32, jnp.bfloat16

    def _dot(a, b):
        return jnp.dot(a.astype(bf), b.astype(bf),
                       preferred_element_type=f32)

    nodes = jnp.concatenate([na_ref[...], nb_ref[...]], -1)
    edges = jnp.concatenate([ea_ref[...], eb_ref[...]], -1)

    def _bd(w, rows):
        z = jnp.zeros((rows, EMB), f32)
        return jnp.concatenate(
            [jnp.concatenate([w, z], 1), jnp.concatenate([z, w], 1)], 0)

    w_src2 = _bd(wpre_ref[:N_F, :], N_F)
    w_dst2 = _bd(wpre_ref[N_F:2 * N_F, :], N_F)
    w_e2 = _bd(wpre_ref[2 * N_F:, :], E_FEAT)
    w_msg2 = _bd(wmsg_ref[...], EMB)
    w_out2 = _bd(wout_ref[...], EMB)
    sel = _bd(jnp.ones((E_FEAT, EMB), f32), E_FEAT)

    n_e = P_TILE * N_N * N_N
    shp = (P_TILE, N_N, N_N, 2 * EMB)
    adj_e = _dot(edges.reshape(n_e, 2 * E_FEAT), sel).reshape(shp)
    adj_t_e = jnp.swapaxes(adj_e, 1, 2)

    a_src = _dot(nodes.reshape(P_TILE * N_N, 2 * N_F), w_src2).reshape(
        P_TILE, N_N, 2 * EMB)
    a_dst = _dot(nodes.reshape(P_TILE * N_N, 2 * N_F), w_dst2).reshape(
        P_TILE, N_N, 2 * EMB)
    e_term = _dot(edges.reshape(n_e, 2 * E_FEAT), w_e2).reshape(shp)
    ef = jnp.tanh(a_src[:, :, None, :] + a_dst[:, None, :, :] + e_term)
    ef_t = jnp.swapaxes(ef, 1, 2)

    def _mm(x):
        return _dot(x.reshape(n_e, 2 * EMB), w_msg2).reshape(shp)

    def _pass(x_bf, mask_e, ef_out, reduce_axis):
        xr = x_bf.astype(f32)
        masked = xr * mask_e
        s = masked.sum(axis=reduce_axis, keepdims=True)
        return jnp.tanh(ef_out + _mm(s - masked))

    y = jnp.tanh(ef_t).astype(bf)
    x = _pass(y, adj_t_e, ef, reduce_axis=1).astype(bf)
    y = _pass(x, adj_e, ef_t, reduce_axis=2).astype(bf)
    x = _pass(y, adj_t_e, ef, reduce_axis=1)

    graph = (x * adj_e).sum(axis=1).sum(axis=1)
    out = jnp.tanh(_dot(graph, w_out2))
    oa_ref[0, :, :] = out[:, :EMB]
    ob_ref[0, :, :] = out[:, EMB:]


def kernel(nodes, edges, W_pre, W_msg, W_out):
    grid = (HALF // P_TILE,)
    shift = HALF // P_TILE
    out_a, out_b = pl.pallas_call(
        _body,
        grid=grid,
        in_specs=[
            pl.BlockSpec((P_TILE, N_N, N_F), lambda i: (i, 0, 0)),
            pl.BlockSpec((P_TILE, N_N, N_F), lambda i: (i + shift, 0, 0)),
            pl.BlockSpec((P_TILE, N_N, N_N, E_FEAT), lambda i: (i, 0, 0, 0)),
            pl.BlockSpec((P_TILE, N_N, N_N, E_FEAT),
                         lambda i: (i + shift, 0, 0, 0)),
            pl.BlockSpec((2 * N_F + E_FEAT, EMB), lambda i: (0, 0)),
            pl.BlockSpec((EMB, EMB), lambda i: (0, 0)),
            pl.BlockSpec((EMB, EMB), lambda i: (0, 0)),
        ],
        out_specs=[
            pl.BlockSpec((1, P_TILE, EMB), lambda i: (i, 0, 0)),
            pl.BlockSpec((1, P_TILE, EMB), lambda i: (i, 0, 0)),
        ],
        out_shape=[
            jax.ShapeDtypeStruct((shift, P_TILE, EMB), jnp.float32),
            jax.ShapeDtypeStruct((shift, P_TILE, EMB), jnp.float32),
        ],
        compiler_params=pltpu.CompilerParams(
            dimension_semantics=("parallel",)),
    )(nodes, nodes, edges, edges, W_pre, W_msg, W_out)
    return jnp.concatenate([out_a, out_b], axis=0).reshape(N_G, EMB)
